# Initial kernel scaffold; baseline (speedup 1.0000x reference)
import jax
import jax.numpy as jnp
from jax.experimental import pallas as pl


def kernel(x_a, x_b, edge_index_ab, edge_index_ba, batch_a, batch_b, params):
    raise NotImplementedError("write your pallas kernel here")


# speedup vs baseline: 10.7202x; 10.7202x over previous
import functools
import math

import jax
import jax.numpy as jnp
from jax import lax
from jax.experimental import pallas as pl
from jax.experimental.pallas import tpu as pltpu
from jax.experimental.pallas import tpu_sc as plsc

H = 4
HID = 128
D = HID // H
OUT = 8
NG = 64

N_PAD = 25088
E_PAD = 303104
_SC_CHUNK = 296
_NW = 32


def _sc_gather(table, idx, d):
    b_per_w = E_PAD // _NW
    n_chunks = b_per_w // _SC_CHUNK
    mesh = plsc.VectorSubcoreMesh(core_axis_name="c", subcore_axis_name="s")

    @functools.partial(
        pl.kernel,
        mesh=mesh,
        out_type=jax.ShapeDtypeStruct((E_PAD, d), jnp.float32),
        scratch_types=[
            pltpu.VMEM((_SC_CHUNK,), jnp.int32),
            pltpu.VMEM((_SC_CHUNK, d), jnp.float32),
            pltpu.SemaphoreType.DMA,
        ],
    )
    def k(table_hbm, idx_hbm, out_hbm, idx_v, rows_v, sem):
        wid = lax.axis_index("s") * 2 + lax.axis_index("c")
        base = wid * b_per_w

        def chunk(i, _):
            ofs = base + i * _SC_CHUNK
            pltpu.sync_copy(idx_hbm.at[pl.ds(ofs, _SC_CHUNK)], idx_v)
            pltpu.async_copy(table_hbm.at[idx_v], rows_v, sem).wait()
            pltpu.sync_copy(rows_v, out_hbm.at[pl.ds(ofs, _SC_CHUNK)])
            return ()

        lax.fori_loop(0, n_chunks, chunk, ())

    return k(table, idx)


def _proj(x, w, b):
    blk = 512

    def body(x_ref, w_ref, b_ref, o_ref):
        o_ref[...] = (
            jnp.dot(x_ref[...], w_ref[...], preferred_element_type=jnp.float32)
            + b_ref[...]
        )

    return pl.pallas_call(
        body,
        grid=(x.shape[0] // blk,),
        in_specs=[
            pl.BlockSpec((blk, HID), lambda i: (i, 0)),
            pl.BlockSpec((HID, 3 * HID), lambda i: (0, 0)),
            pl.BlockSpec((1, 3 * HID), lambda i: (0, 0)),
        ],
        out_specs=pl.BlockSpec((blk, 3 * HID), lambda i: (i, 0)),
        out_shape=jax.ShapeDtypeStruct((x.shape[0], 3 * HID), jnp.float32),
    )(x, w, b.reshape(1, -1))


def _edge(qe, kve):
    blk = 1024

    def body(q_ref, kv_ref, w_ref, wv_ref):
        q = q_ref[...]
        k = kv_ref[:, :HID]
        v = kv_ref[:, HID:]
        w = jnp.exp(jnp.sum(q * k, axis=1, keepdims=True))
        w_ref[...] = w
        wv_ref[...] = v * w

    return pl.pallas_call(
        body,
        grid=(E_PAD // blk,),
        in_specs=[
            pl.BlockSpec((blk, HID), lambda i: (i, 0)),
            pl.BlockSpec((blk, 2 * HID), lambda i: (i, 0)),
        ],
        out_specs=[
            pl.BlockSpec((blk, 1), lambda i: (i, 0)),
            pl.BlockSpec((blk, HID), lambda i: (i, 0)),
        ],
        out_shape=[
            jax.ShapeDtypeStruct((E_PAD, 1), jnp.float32),
            jax.ShapeDtypeStruct((E_PAD, HID), jnp.float32),
        ],
    )(qe, kve)


def _post(msg, den, x, aw, ab, mix):
    blk = 512

    def body(m_ref, d_ref, x_ref, w_ref, b_ref, mix_ref, o_ref):
        o = m_ref[...] / (d_ref[...] + 1e-16)
        g = 0.5 * o * (1.0 + lax.erf(o * (1.0 / math.sqrt(2.0))))
        y = (
            jnp.dot(g, w_ref[...], preferred_element_type=jnp.float32)
            + b_ref[...]
        )
        a = mix_ref[0, 0]
        o_ref[...] = jnp.maximum(a * y + (1.0 - a) * x_ref[...], 0.0)

    return pl.pallas_call(
        body,
        grid=(msg.shape[0] // blk,),
        in_specs=[
            pl.BlockSpec((blk, HID), lambda i: (i, 0)),
            pl.BlockSpec((blk, 1), lambda i: (i, 0)),
            pl.BlockSpec((blk, HID), lambda i: (i, 0)),
            pl.BlockSpec((HID, HID), lambda i: (0, 0)),
            pl.BlockSpec((1, HID), lambda i: (0, 0)),
            pl.BlockSpec((1, 1), lambda i: (0, 0)),
        ],
        out_specs=pl.BlockSpec((blk, HID), lambda i: (i, 0)),
        out_shape=jax.ShapeDtypeStruct((msg.shape[0], HID), jnp.float32),
    )(msg, den, x, aw, ab.reshape(1, -1), mix.reshape(1, 1))


def _pool_acc(x_all, b_all):
    blk = 512

    def body(x_ref, b_ref, s_ref, c_ref):
        @pl.when(pl.program_id(0) == 0)
        def _():
            s_ref[...] = jnp.zeros_like(s_ref)
            c_ref[...] = jnp.zeros_like(c_ref)

        labels = lax.broadcasted_iota(jnp.int32, (blk, NG), 1)
        oh = (b_ref[...] == labels).astype(jnp.float32)
        s_ref[...] += lax.dot_general(
            oh, x_ref[...], (((0,), (0,)), ((), ())),
            preferred_element_type=jnp.float32)
        c_ref[...] += jnp.sum(oh, axis=0)[:, None]

    return pl.pallas_call(
        body,
        grid=(x_all.shape[0] // blk,),
        in_specs=[
            pl.BlockSpec((blk, HID), lambda i: (i, 0)),
            pl.BlockSpec((blk, 1), lambda i: (i, 0)),
        ],
        out_specs=[
            pl.BlockSpec((NG, HID), lambda i: (0, 0)),
            pl.BlockSpec((NG, HID), lambda i: (0, 0)),
        ],
        out_shape=[
            jax.ShapeDtypeStruct((NG, HID), jnp.float32),
            jax.ShapeDtypeStruct((NG, HID), jnp.float32),
        ],
    )(x_all, b_all)


def _pool_fin(sums, cnts, lw, lb):
    def body(s_ref, c_ref, w_ref, b_ref, o_ref):
        pooled = s_ref[...] / jnp.maximum(c_ref[...], 1.0)
        o_ref[...] = jnp.tanh(
            jnp.dot(pooled, w_ref[...], preferred_element_type=jnp.float32)
            + b_ref[...]
        )

    return pl.pallas_call(
        body,
        out_shape=jax.ShapeDtypeStruct((NG, OUT), jnp.float32),
    )(sums, cnts, lw, lb.reshape(1, -1))


def _block_diag(rel):
    out = jnp.zeros((HID, HID), jnp.float32)
    for h in range(H):
        out = out.at[h * D:(h + 1) * D, h * D:(h + 1) * D].set(rel[h])
    return out


def _fold_weights(p, nt, et_src):
    ep = p[et_src]
    bd_a = _block_diag(ep["a_rel"])
    bd_m = _block_diag(ep["m_rel"])
    scale = jnp.repeat(ep["p_rel"] / math.sqrt(D), D)
    wq, bq = p[nt]["q"]["w"], p[nt]["q"]["b"]
    wk = (p[nt]["k"]["w"] @ bd_a) * scale[None, :]
    bk = (p[nt]["k"]["b"] @ bd_a) * scale
    wv = p[nt]["v"]["w"] @ bd_m
    bv = p[nt]["v"]["b"] @ bd_m
    w = jnp.concatenate([wq, wk, wv], axis=1)
    b = jnp.concatenate([bq, bk, bv], axis=0)
    return w, b


def _pad_rows(x, n):
    return jnp.pad(x, ((0, n - x.shape[0]), (0, 0)))


def _conv_layer(xa, xb, s_ab, t_ab, s_ba, t_ba, p):
    n_a, n_b = 25000, 25000
    wa, ba = _fold_weights(p, "a", "rel_a_b")
    wb, bb = _fold_weights(p, "b", "rel_b_a")
    proj_a = _proj(xa, wa, ba)
    proj_b = _proj(xb, wb, bb)

    outs = {}
    for (src_p, dst_p, dst_x, s_idx, t_idx, dst_nt, n_dst) in (
        (proj_a, proj_b, xb, s_ab, t_ab, "b", n_b),
        (proj_b, proj_a, xa, s_ba, t_ba, "a", n_a),
    ):
        kv_e = _sc_gather(src_p[:, HID:], s_idx, 2 * HID)
        q_e = _sc_gather(dst_p[:, :HID], t_idx, HID)
        w_e, wv_e = _edge(q_e, kv_e)
        e = s_idx.shape[0]
        t_true = t_idx[:300000]
        den = jax.ops.segment_sum(w_e[:300000, 0], t_true, num_segments=n_dst)
        msg = jax.ops.segment_sum(wv_e[:300000], t_true, num_segments=n_dst)
        mix = jax.nn.sigmoid(p[dst_nt]["skip"])
        outs[dst_nt] = _post(
            _pad_rows(msg, N_PAD),
            _pad_rows(den[:, None], N_PAD),
            dst_x,
            p[dst_nt]["a"]["w"],
            p[dst_nt]["a"]["b"],
            mix,
        )
    return outs["a"], outs["b"]


def kernel(x_a, x_b, edge_index_ab, edge_index_ba, batch_a, batch_b, params):
    xa = _pad_rows(x_a, N_PAD)
    xb = _pad_rows(x_b, N_PAD)

    def pad_idx(v):
        return jnp.pad(v.astype(jnp.int32), (0, E_PAD - v.shape[0]))

    s_ab, t_ab = pad_idx(edge_index_ab[0]), pad_idx(edge_index_ab[1])
    s_ba, t_ba = pad_idx(edge_index_ba[0]), pad_idx(edge_index_ba[1])

    ha, hb = _conv_layer(xa, xb, s_ab, t_ab, s_ba, t_ba, params["conv1"])
    ha, hb = _conv_layer(ha, hb, s_ab, t_ab, s_ba, t_ba, params["conv2"])

    pad_g = N_PAD - 25000
    b_all = jnp.concatenate([
        batch_a.astype(jnp.int32), jnp.full((pad_g,), NG, jnp.int32),
        batch_b.astype(jnp.int32), jnp.full((pad_g,), NG, jnp.int32),
    ])[:, None]
    x_all = jnp.concatenate([ha, hb], axis=0)
    sums, cnts = _pool_acc(x_all, b_all)
    return _pool_fin(sums, cnts, params["lin"]["w"], params["lin"]["b"])
